# baseline (speedup 1.0000x reference)
import functools
import math

import jax
import jax.numpy as jnp
from jax import lax
from jax.experimental import pallas as pl
from jax.experimental.pallas import tpu as pltpu
from jax.experimental.pallas import tpu_sc as plsc

_V = 100000
_LS = 0.1
_CONF = 1.0 - _LS
_NUM_STEPS = 100000.0
_TOPK = 100
_PCT = 0.05
_START_SMOOTH = _LS / (_V - 2)
_END_SMOOTH = (1.0 - _PCT) * _LS / (_V - 2 - _TOPK)
_STEP_SIZE = (_END_SMOOTH - _START_SMOOTH) / _NUM_STEPS
_TOPK_START = _LS / (_V - 2)
_TOPK_END = _PCT * _LS / _TOPK
_TOPK_STEP = (_TOPK_END - _TOPK_START) / _NUM_STEPS
_CLOGC = _CONF * math.log(_CONF)

_KP = 128
_NCONC = 100
_CB = 2048


def _sc_gather(flat, cols):
    B, KP = cols.shape
    info = plsc.get_sparse_core_info()
    nw = info.num_cores * info.num_subcores
    rpw = B // nw
    mesh = plsc.VectorSubcoreMesh(core_axis_name="c", subcore_axis_name="s")

    @functools.partial(
        pl.kernel,
        mesh=mesh,
        out_type=jax.ShapeDtypeStruct((B, KP), jnp.float32),
        scratch_types=[
            pltpu.VMEM((rpw, KP), jnp.int32),
            pltpu.VMEM((rpw, KP), jnp.int32),
            pltpu.VMEM((rpw, KP), jnp.float32),
            pltpu.SemaphoreType.DMA,
        ],
    )
    def gk(flat_hbm, cols_hbm, vals_hbm, cols_v, idx_v, vals_v, sem):
        wid = lax.axis_index("s") * info.num_cores + lax.axis_index("c")
        base = wid * rpw
        pltpu.sync_copy(cols_hbm.at[pl.ds(base, rpw)], cols_v)
        copies = []
        for j in range(rpw):
            rv = (base + j) * _V
            for q in range(KP // 16):
                idx_v[j, pl.ds(q * 16, 16)] = (
                    cols_v[j, pl.ds(q * 16, 16)] + rv)
            copies.append(
                pltpu.async_copy(flat_hbm.at[idx_v.at[j]], vals_v.at[j], sem))
        for c in copies:
            c.wait()
        pltpu.sync_copy(vals_v, vals_hbm.at[pl.ds(base, rpw)])

    return gk(flat, cols)


_RB = 32


def _tc_rowsum(output):
    B, V = output.shape
    nfull = V // 128
    tail = V - nfull * 128

    def body(out_blk, acc_ref):
        part = jnp.zeros((_RB, 128), jnp.float32)
        for kk in range(nfull):
            part = part + out_blk[:, pl.ds(kk * 128, 128)]
        if tail:
            part = part + jnp.concatenate(
                [out_blk[:, pl.ds(nfull * 128, tail)],
                 jnp.zeros((_RB, 128 - tail), jnp.float32)], axis=1)
        acc_ref[...] = part

    return pl.pallas_call(
        body,
        grid=(B // _RB,),
        in_specs=[pl.BlockSpec((_RB, V), lambda i: (i, 0))],
        out_specs=pl.BlockSpec((_RB, 128), lambda i: (i, 0)),
        out_shape=jax.ShapeDtypeStruct((B, 128), jnp.float32),
    )(output)


def _tc_combine(acc, cols, vals, params):
    B = acc.shape[0]

    def body(acc_ref, cols_ref, vals_ref, par_ref, out_ref):
        fill = par_ref[0, 0]
        topk = par_ref[0, 1]
        logf = par_ref[0, 2]
        logt = par_ref[0, 3]
        colsa = cols_ref[...]
        valsa = vals_ref[...]
        lane = lax.broadcasted_iota(jnp.int32, (B, _KP), 1)
        t = jnp.sum(jnp.where(lane == _NCONC, colsa, 0),
                    axis=1, keepdims=True)
        tv = jnp.sum(jnp.where(lane == _NCONC, valsa, 0.0),
                     axis=1, keepdims=True)
        zv = jnp.sum(jnp.where(lane == _NCONC + 1, valsa, 0.0),
                     axis=1, keepdims=True)
        dup = jnp.zeros((B, _KP), jnp.bool_)
        for s in range(1, _NCONC):
            shifted = jnp.concatenate(
                [jnp.full((B, s), -1, jnp.int32), colsa[:, :_KP - s]], axis=1)
            dup = dup | (colsa == shifted)
        keptf = (jnp.where(dup, 0.0, 1.0)
                 * jnp.where(lane < _NCONC, 1.0, 0.0)
                 * jnp.where(colsa != t, 1.0, 0.0))
        d = jnp.sum(keptf, axis=1, keepdims=True)
        zin = jnp.sum(keptf * jnp.where(colsa == 0, 1.0, 0.0),
                      axis=1, keepdims=True)
        gsum = jnp.sum(keptf * valsa, axis=1, keepdims=True)
        srow = jnp.sum(acc_ref[...], axis=1, keepdims=True)
        active = jnp.where(t != 0, 1.0, 0.0)
        plogp = (_CLOGC + d * topk * logt
                 + (_V - 2.0 - d + zin) * fill * logf)
        pdot = (fill * srow + (_CONF - fill) * tv + (topk - fill) * gsum
                - (1.0 - zin) * fill * zv)
        out_ref[0, 0] = jnp.sum(active * (plogp - pdot))

    return pl.pallas_call(
        body,
        grid=(1,),
        in_specs=[
            pl.BlockSpec((B, 128), lambda i: (0, 0)),
            pl.BlockSpec((B, _KP), lambda i: (0, 0)),
            pl.BlockSpec((B, _KP), lambda i: (0, 0)),
            pl.BlockSpec((8, 128), lambda i: (0, 0)),
        ],
        out_specs=pl.BlockSpec(memory_space=pltpu.SMEM),
        out_shape=jax.ShapeDtypeStruct((1, 1), jnp.float32),
    )(acc, cols, vals, params)


def kernel(output, target, concepts, batch_idx):
    B, V = output.shape
    k = concepts.shape[1]
    bi = jnp.asarray(batch_idx, jnp.float32)
    fill = _START_SMOOTH + bi * _STEP_SIZE
    topk = _TOPK_START + bi * _TOPK_STEP
    params = (jnp.zeros((8, 128), jnp.float32)
              .at[0, 0].set(fill)
              .at[0, 1].set(topk)
              .at[0, 2].set(jnp.log(fill))
              .at[0, 3].set(jnp.log(topk)))
    cols = jnp.concatenate(
        [concepts.astype(jnp.int32),
         target.astype(jnp.int32)[:, None],
         jnp.zeros((B, _KP - k - 1), jnp.int32)], axis=1)
    vals = _sc_gather(output.reshape(B * V), cols)
    acc = _tc_rowsum(output)
    total = _tc_combine(acc, cols, vals, params)
    return total[0, 0]


# speedup vs baseline: 1.1362x; 1.1362x over previous
import functools
import math

import jax
import jax.numpy as jnp
from jax import lax
from jax.experimental import pallas as pl
from jax.experimental.pallas import tpu as pltpu
from jax.experimental.pallas import tpu_sc as plsc

_V = 100000
_VT = 782
_VP = _VT * 128
_LS = 0.1
_CONF = 1.0 - _LS
_NUM_STEPS = 100000.0
_TOPK = 100
_PCT = 0.05
_START_SMOOTH = _LS / (_V - 2)
_END_SMOOTH = (1.0 - _PCT) * _LS / (_V - 2 - _TOPK)
_STEP_SIZE = (_END_SMOOTH - _START_SMOOTH) / _NUM_STEPS
_TOPK_START = _LS / (_V - 2)
_TOPK_END = _PCT * _LS / _TOPK
_TOPK_STEP = (_TOPK_END - _TOPK_START) / _NUM_STEPS
_CLOGC = _CONF * math.log(_CONF)

_KP = 128
_NCONC = 100
_RB = 8
_NBUF = 8


def _tc_stream(output):
    B, V = output.shape
    nfull = V // 128
    tail = V - nfull * 128
    nstep = B // _RB

    def body(out_hbm, acc_ref, flatp_ref, bufs, sems):
        i = pl.program_id(0)

        def fetch(blk, slot):
            pltpu.make_async_copy(
                out_hbm.at[pl.ds(blk * _RB, _RB)],
                bufs.at[slot], sems.at[slot]).start()

        @pl.when(i == 0)
        def _():
            for b in range(_NBUF):
                fetch(b, b)

        slot = lax.rem(i, _NBUF)
        pltpu.make_async_copy(
            out_hbm.at[pl.ds(i * _RB, _RB)],
            bufs.at[slot], sems.at[slot]).wait()
        buf = bufs.at[slot]
        part = jnp.zeros((_RB, 128), jnp.float32)
        for kk in range(nfull):
            x = buf[:, pl.ds(kk * 128, 128)]
            part = part + x
            flatp_ref[:, kk, :] = x
        xt = jnp.concatenate(
            [buf[:, pl.ds(nfull * 128, tail)],
             jnp.zeros((_RB, 128 - tail), jnp.float32)], axis=1)
        part = part + xt
        flatp_ref[:, nfull, :] = xt
        acc_ref[...] = part

        @pl.when(i + _NBUF < nstep)
        def _():
            fetch(i + _NBUF, slot)

    return pl.pallas_call(
        body,
        grid=(nstep,),
        in_specs=[pl.BlockSpec(memory_space=pl.ANY)],
        out_specs=[
            pl.BlockSpec((_RB, 128), lambda i: (i, 0)),
            pl.BlockSpec((_RB, _VT, 128), lambda i: (i, 0, 0)),
        ],
        out_shape=[
            jax.ShapeDtypeStruct((B, 128), jnp.float32),
            jax.ShapeDtypeStruct((B, _VT, 128), jnp.float32),
        ],
        scratch_shapes=[
            pltpu.VMEM((_NBUF, _RB, V), jnp.float32),
            pltpu.SemaphoreType.DMA((_NBUF,)),
        ],
    )(output)


def _sc_gather(flat, cols):
    B, KP = cols.shape
    info = plsc.get_sparse_core_info()
    nw = info.num_cores * info.num_subcores
    rpw = B // nw
    mesh = plsc.VectorSubcoreMesh(core_axis_name="c", subcore_axis_name="s")

    @functools.partial(
        pl.kernel,
        mesh=mesh,
        out_type=jax.ShapeDtypeStruct((B, KP), jnp.float32),
        scratch_types=[
            pltpu.VMEM((rpw, KP), jnp.int32),
            pltpu.VMEM((rpw, KP), jnp.int32),
            pltpu.VMEM((rpw, KP), jnp.float32),
            pltpu.SemaphoreType.DMA,
        ],
    )
    def gk(flat_hbm, cols_hbm, vals_hbm, cols_v, idx_v, vals_v, sem):
        wid = lax.axis_index("s") * info.num_cores + lax.axis_index("c")
        base = wid * rpw
        pltpu.sync_copy(cols_hbm.at[pl.ds(base, rpw)], cols_v)
        copies = []
        for j in range(rpw):
            rv = (base + j) * _VP
            for q in range(KP // 16):
                idx_v[j, pl.ds(q * 16, 16)] = (
                    cols_v[j, pl.ds(q * 16, 16)] + rv)
            copies.append(
                pltpu.async_copy(flat_hbm.at[idx_v.at[j]], vals_v.at[j], sem))
        for c in copies:
            c.wait()
        pltpu.sync_copy(vals_v, vals_hbm.at[pl.ds(base, rpw)])

    return gk(flat, cols)


def _tc_combine(acc, cols, vals, params):
    B = acc.shape[0]

    def body(acc_ref, cols_ref, vals_ref, par_ref, out_ref):
        fill = par_ref[0, 0]
        topk = par_ref[0, 1]
        logf = par_ref[0, 2]
        logt = par_ref[0, 3]
        colsa = cols_ref[...]
        valsa = vals_ref[...]
        lane = lax.broadcasted_iota(jnp.int32, (B, _KP), 1)
        t = jnp.sum(jnp.where(lane == _NCONC, colsa, 0),
                    axis=1, keepdims=True)
        tv = jnp.sum(jnp.where(lane == _NCONC, valsa, 0.0),
                     axis=1, keepdims=True)
        zv = jnp.sum(jnp.where(lane == _NCONC + 1, valsa, 0.0),
                     axis=1, keepdims=True)
        dup = jnp.zeros((B, _KP), jnp.bool_)
        for s in range(1, _NCONC):
            shifted = jnp.concatenate(
                [jnp.full((B, s), -1, jnp.int32), colsa[:, :_KP - s]], axis=1)
            dup = dup | (colsa == shifted)
        keptf = (jnp.where(dup, 0.0, 1.0)
                 * jnp.where(lane < _NCONC, 1.0, 0.0)
                 * jnp.where(colsa != t, 1.0, 0.0))
        d = jnp.sum(keptf, axis=1, keepdims=True)
        zin = jnp.sum(keptf * jnp.where(colsa == 0, 1.0, 0.0),
                      axis=1, keepdims=True)
        gsum = jnp.sum(keptf * valsa, axis=1, keepdims=True)
        srow = jnp.sum(acc_ref[...], axis=1, keepdims=True)
        active = jnp.where(t != 0, 1.0, 0.0)
        plogp = (_CLOGC + d * topk * logt
                 + (_V - 2.0 - d + zin) * fill * logf)
        pdot = (fill * srow + (_CONF - fill) * tv + (topk - fill) * gsum
                - (1.0 - zin) * fill * zv)
        out_ref[0, 0] = jnp.sum(active * (plogp - pdot))

    return pl.pallas_call(
        body,
        grid=(1,),
        in_specs=[
            pl.BlockSpec((B, 128), lambda i: (0, 0)),
            pl.BlockSpec((B, _KP), lambda i: (0, 0)),
            pl.BlockSpec((B, _KP), lambda i: (0, 0)),
            pl.BlockSpec((8, 128), lambda i: (0, 0)),
        ],
        out_specs=pl.BlockSpec(memory_space=pltpu.SMEM),
        out_shape=jax.ShapeDtypeStruct((1, 1), jnp.float32),
    )(acc, cols, vals, params)


def kernel(output, target, concepts, batch_idx):
    B, V = output.shape
    k = concepts.shape[1]
    bi = jnp.asarray(batch_idx, jnp.float32)
    fill = _START_SMOOTH + bi * _STEP_SIZE
    topk = _TOPK_START + bi * _TOPK_STEP
    params = (jnp.zeros((8, 128), jnp.float32)
              .at[0, 0].set(fill)
              .at[0, 1].set(topk)
              .at[0, 2].set(jnp.log(fill))
              .at[0, 3].set(jnp.log(topk)))
    cols = jnp.concatenate(
        [concepts.astype(jnp.int32),
         target.astype(jnp.int32)[:, None],
         jnp.zeros((B, _KP - k - 1), jnp.int32)], axis=1)
    acc, flatp = _tc_stream(output)
    vals = _sc_gather(flatp.reshape(B * _VP), cols)
    total = _tc_combine(acc, cols, vals, params)
    return total[0, 0]


# speedup vs baseline: 1.1394x; 1.0028x over previous
import functools
import math

import jax
import jax.numpy as jnp
from jax import lax
from jax.experimental import pallas as pl
from jax.experimental.pallas import tpu as pltpu
from jax.experimental.pallas import tpu_sc as plsc

_V = 100000
_VT = 782
_VP = _VT * 128
_LS = 0.1
_CONF = 1.0 - _LS
_NUM_STEPS = 100000.0
_TOPK = 100
_PCT = 0.05
_START_SMOOTH = _LS / (_V - 2)
_END_SMOOTH = (1.0 - _PCT) * _LS / (_V - 2 - _TOPK)
_STEP_SIZE = (_END_SMOOTH - _START_SMOOTH) / _NUM_STEPS
_TOPK_START = _LS / (_V - 2)
_TOPK_END = _PCT * _LS / _TOPK
_TOPK_STEP = (_TOPK_END - _TOPK_START) / _NUM_STEPS
_CLOGC = _CONF * math.log(_CONF)

_KP = 128
_NCONC = 100
_RB = 8
_NBUF = 6


def _tc_stream(output):
    B, V = output.shape
    nfull = V // 128
    tail = V - nfull * 128
    nstep = B // _RB

    def body(out_hbm, acc_ref, flatp_hbm, bufs, fbufs, rsems, wsems):
        i = pl.program_id(0)

        def fetch(blk, slot):
            pltpu.make_async_copy(
                out_hbm.at[pl.ds(blk * _RB, _RB)],
                bufs.at[slot], sems_r.at[slot]).start()

        def wdesc(blk, slot):
            return pltpu.make_async_copy(
                fbufs.at[slot],
                flatp_hbm.at[pl.ds(blk * _RB, _RB)], wsems.at[slot])

        sems_r = rsems

        @pl.when(i == 0)
        def _():
            for b in range(_NBUF):
                fetch(b, b)

        slot = lax.rem(i, _NBUF)
        pltpu.make_async_copy(
            out_hbm.at[pl.ds(i * _RB, _RB)],
            bufs.at[slot], rsems.at[slot]).wait()

        @pl.when(i >= _NBUF)
        def _():
            wdesc(i - _NBUF, slot).wait()

        buf = bufs.at[slot]
        fb = fbufs.at[slot]
        part = jnp.zeros((_RB, 128), jnp.float32)
        for kk in range(nfull):
            x = buf[:, pl.ds(kk * 128, 128)]
            part = part + x
            fb[:, kk, :] = x
        xt = jnp.concatenate(
            [buf[:, pl.ds(nfull * 128, tail)],
             jnp.zeros((_RB, 128 - tail), jnp.float32)], axis=1)
        part = part + xt
        fb[:, nfull, :] = xt
        acc_ref[...] = part
        wdesc(i, slot).start()

        @pl.when(i + _NBUF < nstep)
        def _():
            fetch(i + _NBUF, slot)

        @pl.when(i == nstep - 1)
        def _():
            for b in range(_NBUF):
                blk = nstep - _NBUF + b
                wdesc(blk, blk % _NBUF).wait()

    return pl.pallas_call(
        body,
        grid=(nstep,),
        in_specs=[pl.BlockSpec(memory_space=pl.ANY)],
        out_specs=[
            pl.BlockSpec((_RB, 128), lambda i: (i, 0)),
            pl.BlockSpec(memory_space=pl.ANY),
        ],
        out_shape=[
            jax.ShapeDtypeStruct((B, 128), jnp.float32),
            jax.ShapeDtypeStruct((B, _VT, 128), jnp.float32),
        ],
        scratch_shapes=[
            pltpu.VMEM((_NBUF, _RB, V), jnp.float32),
            pltpu.VMEM((_NBUF, _RB, _VT, 128), jnp.float32),
            pltpu.SemaphoreType.DMA((_NBUF,)),
            pltpu.SemaphoreType.DMA((_NBUF,)),
        ],
    )(output)


def _sc_gather(flat, cols):
    B, KP = cols.shape
    info = plsc.get_sparse_core_info()
    nw = info.num_cores * info.num_subcores
    rpw = B // nw
    mesh = plsc.VectorSubcoreMesh(core_axis_name="c", subcore_axis_name="s")

    @functools.partial(
        pl.kernel,
        mesh=mesh,
        out_type=jax.ShapeDtypeStruct((B, KP), jnp.float32),
        scratch_types=[
            pltpu.VMEM((rpw, KP), jnp.int32),
            pltpu.VMEM((rpw, KP), jnp.int32),
            pltpu.VMEM((rpw, KP), jnp.float32),
            pltpu.SemaphoreType.DMA,
        ],
    )
    def gk(flat_hbm, cols_hbm, vals_hbm, cols_v, idx_v, vals_v, sem):
        wid = lax.axis_index("s") * info.num_cores + lax.axis_index("c")
        base = wid * rpw
        pltpu.sync_copy(cols_hbm.at[pl.ds(base, rpw)], cols_v)
        copies = []
        for j in range(rpw):
            rv = (base + j) * _VP
            for q in range(KP // 16):
                idx_v[j, pl.ds(q * 16, 16)] = (
                    cols_v[j, pl.ds(q * 16, 16)] + rv)
            copies.append(
                pltpu.async_copy(flat_hbm.at[idx_v.at[j]], vals_v.at[j], sem))
        for c in copies:
            c.wait()
        pltpu.sync_copy(vals_v, vals_hbm.at[pl.ds(base, rpw)])

    return gk(flat, cols)


def _tc_combine(acc, cols, vals, params):
    B = acc.shape[0]

    def body(acc_ref, cols_ref, vals_ref, par_ref, out_ref):
        fill = par_ref[0, 0]
        topk = par_ref[0, 1]
        logf = par_ref[0, 2]
        logt = par_ref[0, 3]
        colsa = cols_ref[...]
        valsa = vals_ref[...]
        lane = lax.broadcasted_iota(jnp.int32, (B, _KP), 1)
        t = jnp.sum(jnp.where(lane == _NCONC, colsa, 0),
                    axis=1, keepdims=True)
        tv = jnp.sum(jnp.where(lane == _NCONC, valsa, 0.0),
                     axis=1, keepdims=True)
        zv = jnp.sum(jnp.where(lane == _NCONC + 1, valsa, 0.0),
                     axis=1, keepdims=True)
        dup = jnp.zeros((B, _KP), jnp.bool_)
        for s in range(1, _NCONC):
            shifted = jnp.concatenate(
                [jnp.full((B, s), -1, jnp.int32), colsa[:, :_KP - s]], axis=1)
            dup = dup | (colsa == shifted)
        keptf = (jnp.where(dup, 0.0, 1.0)
                 * jnp.where(lane < _NCONC, 1.0, 0.0)
                 * jnp.where(colsa != t, 1.0, 0.0))
        d = jnp.sum(keptf, axis=1, keepdims=True)
        zin = jnp.sum(keptf * jnp.where(colsa == 0, 1.0, 0.0),
                      axis=1, keepdims=True)
        gsum = jnp.sum(keptf * valsa, axis=1, keepdims=True)
        srow = jnp.sum(acc_ref[...], axis=1, keepdims=True)
        active = jnp.where(t != 0, 1.0, 0.0)
        plogp = (_CLOGC + d * topk * logt
                 + (_V - 2.0 - d + zin) * fill * logf)
        pdot = (fill * srow + (_CONF - fill) * tv + (topk - fill) * gsum
                - (1.0 - zin) * fill * zv)
        out_ref[0, 0] = jnp.sum(active * (plogp - pdot))

    return pl.pallas_call(
        body,
        grid=(1,),
        in_specs=[
            pl.BlockSpec((B, 128), lambda i: (0, 0)),
            pl.BlockSpec((B, _KP), lambda i: (0, 0)),
            pl.BlockSpec((B, _KP), lambda i: (0, 0)),
            pl.BlockSpec((8, 128), lambda i: (0, 0)),
        ],
        out_specs=pl.BlockSpec(memory_space=pltpu.SMEM),
        out_shape=jax.ShapeDtypeStruct((1, 1), jnp.float32),
    )(acc, cols, vals, params)


def kernel(output, target, concepts, batch_idx):
    B, V = output.shape
    k = concepts.shape[1]
    bi = jnp.asarray(batch_idx, jnp.float32)
    fill = _START_SMOOTH + bi * _STEP_SIZE
    topk = _TOPK_START + bi * _TOPK_STEP
    params = (jnp.zeros((8, 128), jnp.float32)
              .at[0, 0].set(fill)
              .at[0, 1].set(topk)
              .at[0, 2].set(jnp.log(fill))
              .at[0, 3].set(jnp.log(topk)))
    cols = jnp.concatenate(
        [concepts.astype(jnp.int32),
         target.astype(jnp.int32)[:, None],
         jnp.zeros((B, _KP - k - 1), jnp.int32)], axis=1)
    acc, flatp = _tc_stream(output)
    vals = _sc_gather(flatp.reshape(B * _VP), cols)
    total = _tc_combine(acc, cols, vals, params)
    return total[0, 0]
